# Initial kernel scaffold; baseline (speedup 1.0000x reference)
import jax
import jax.numpy as jnp
from jax.experimental import pallas as pl


def kernel(inputs, embedding, ln_scale, ln_bias):
    raise NotImplementedError("write your pallas kernel here")


# speedup vs baseline: 1.7699x; 1.7699x over previous
import jax
import jax.numpy as jnp
from jax.experimental import pallas as pl

BATCH = 4
N_INSTANCE = 1
LENGTH = 2048
HIDDEN = 1024
ROW_TILE = 256


def _ln_body(x_ref, e_ref, s_ref, b_ref, o_ref):
    x = x_ref[0, 0] + e_ref[...]
    mean = jnp.mean(x, axis=-1, keepdims=True)
    xc = x - mean
    var = jnp.mean(xc * xc, axis=-1, keepdims=True)
    y = xc * jax.lax.rsqrt(var + 1e-6)
    o_ref[0, 0] = y * s_ref[0] + b_ref[0]


def kernel(inputs, embedding, ln_scale, ln_bias):
    grid = (LENGTH // ROW_TILE, BATCH)
    return pl.pallas_call(
        _ln_body,
        grid=grid,
        in_specs=[
            pl.BlockSpec((1, 1, ROW_TILE, HIDDEN), lambda l, b: (b, 0, l, 0)),
            pl.BlockSpec((ROW_TILE, HIDDEN), lambda l, b: (l, 0)),
            pl.BlockSpec((1, HIDDEN), lambda l, b: (0, 0)),
            pl.BlockSpec((1, HIDDEN), lambda l, b: (0, 0)),
        ],
        out_specs=pl.BlockSpec((1, 1, ROW_TILE, HIDDEN), lambda l, b: (b, 0, l, 0)),
        out_shape=jax.ShapeDtypeStruct((BATCH, N_INSTANCE, LENGTH, HIDDEN), jnp.float32),
    )(inputs, embedding, ln_scale.reshape(1, HIDDEN), ln_bias.reshape(1, HIDDEN))


# speedup vs baseline: 2.2388x; 1.2649x over previous
import jax
import jax.numpy as jnp
from jax.experimental import pallas as pl

BATCH = 4
N_INSTANCE = 1
LENGTH = 2048
HIDDEN = 1024
ROW_TILE = 512


def _ln_body(x_ref, e_ref, s_ref, b_ref, o_ref):
    x = x_ref[0, 0] + e_ref[...]
    inv_n = 1.0 / HIDDEN
    mean = jnp.sum(x, axis=-1, keepdims=True) * inv_n
    msq = jnp.sum(x * x, axis=-1, keepdims=True) * inv_n
    var = msq - mean * mean
    r = jax.lax.rsqrt(var + 1e-6)
    scale = r * s_ref[0]
    shift = b_ref[0] - (r * mean) * s_ref[0]
    o_ref[0, 0] = x * scale + shift


def kernel(inputs, embedding, ln_scale, ln_bias):
    grid = (LENGTH // ROW_TILE, BATCH)
    return pl.pallas_call(
        _ln_body,
        grid=grid,
        in_specs=[
            pl.BlockSpec((1, 1, ROW_TILE, HIDDEN), lambda l, b: (b, 0, l, 0)),
            pl.BlockSpec((ROW_TILE, HIDDEN), lambda l, b: (l, 0)),
            pl.BlockSpec((1, HIDDEN), lambda l, b: (0, 0)),
            pl.BlockSpec((1, HIDDEN), lambda l, b: (0, 0)),
        ],
        out_specs=pl.BlockSpec((1, 1, ROW_TILE, HIDDEN), lambda l, b: (b, 0, l, 0)),
        out_shape=jax.ShapeDtypeStruct((BATCH, N_INSTANCE, LENGTH, HIDDEN), jnp.float32),
    )(inputs, embedding, ln_scale.reshape(1, HIDDEN), ln_bias.reshape(1, HIDDEN))


# speedup vs baseline: 2.5150x; 1.1234x over previous
import jax
import jax.numpy as jnp
from jax.experimental import pallas as pl

BATCH = 4
N_INSTANCE = 1
LENGTH = 2048
HIDDEN = 1024
ROW_TILE = 1024


def _ln_body(x_ref, e_ref, s_ref, b_ref, o_ref):
    x = x_ref[0, 0] + e_ref[...]
    inv_n = 1.0 / HIDDEN
    mean = jnp.sum(x, axis=-1, keepdims=True) * inv_n
    msq = jnp.sum(x * x, axis=-1, keepdims=True) * inv_n
    var = msq - mean * mean
    r = jax.lax.rsqrt(var + 1e-6)
    scale = r * s_ref[0]
    shift = b_ref[0] - (r * mean) * s_ref[0]
    o_ref[0, 0] = x * scale + shift


def kernel(inputs, embedding, ln_scale, ln_bias):
    grid = (LENGTH // ROW_TILE, BATCH)
    return pl.pallas_call(
        _ln_body,
        grid=grid,
        in_specs=[
            pl.BlockSpec((1, 1, ROW_TILE, HIDDEN), lambda l, b: (b, 0, l, 0)),
            pl.BlockSpec((ROW_TILE, HIDDEN), lambda l, b: (l, 0)),
            pl.BlockSpec((1, HIDDEN), lambda l, b: (0, 0)),
            pl.BlockSpec((1, HIDDEN), lambda l, b: (0, 0)),
        ],
        out_specs=pl.BlockSpec((1, 1, ROW_TILE, HIDDEN), lambda l, b: (b, 0, l, 0)),
        out_shape=jax.ShapeDtypeStruct((BATCH, N_INSTANCE, LENGTH, HIDDEN), jnp.float32),
    )(inputs, embedding, ln_scale.reshape(1, HIDDEN), ln_bias.reshape(1, HIDDEN))


# speedup vs baseline: 2.5942x; 1.0315x over previous
import jax
import jax.numpy as jnp
from jax.experimental import pallas as pl

BATCH = 4
N_INSTANCE = 1
LENGTH = 2048
HIDDEN = 1024
ROW_TILE = 2048


def _ln_body(x_ref, e_ref, s_ref, b_ref, o_ref):
    x = x_ref[0, 0] + e_ref[...]
    inv_n = 1.0 / HIDDEN
    mean = jnp.sum(x, axis=-1, keepdims=True) * inv_n
    msq = jnp.sum(x * x, axis=-1, keepdims=True) * inv_n
    var = msq - mean * mean
    r = jax.lax.rsqrt(var + 1e-6)
    scale = r * s_ref[0]
    shift = b_ref[0] - (r * mean) * s_ref[0]
    o_ref[0, 0] = x * scale + shift


def kernel(inputs, embedding, ln_scale, ln_bias):
    grid = (LENGTH // ROW_TILE, BATCH)
    return pl.pallas_call(
        _ln_body,
        grid=grid,
        in_specs=[
            pl.BlockSpec((1, 1, ROW_TILE, HIDDEN), lambda l, b: (b, 0, l, 0)),
            pl.BlockSpec((ROW_TILE, HIDDEN), lambda l, b: (l, 0)),
            pl.BlockSpec((1, HIDDEN), lambda l, b: (0, 0)),
            pl.BlockSpec((1, HIDDEN), lambda l, b: (0, 0)),
        ],
        out_specs=pl.BlockSpec((1, 1, ROW_TILE, HIDDEN), lambda l, b: (b, 0, l, 0)),
        out_shape=jax.ShapeDtypeStruct((BATCH, N_INSTANCE, LENGTH, HIDDEN), jnp.float32),
    )(inputs, embedding, ln_scale.reshape(1, HIDDEN), ln_bias.reshape(1, HIDDEN))
